# baseline (speedup 1.0000x reference)
import jax
import jax.numpy as jnp
from jax.experimental import pallas as pl
from jax.experimental.pallas import tpu as pltpu

_TOP_K = 10
_TAU = 1e-4
_EPS = 1e-8
_N = 2048
_B = 8
_BLK = 256
_STEPS = _N // _BLK
_CH = 512


def _body(r_ref, rt_ref, m_ref, out_ref, wt_ref, acc_ref):
    step = pl.program_id(0)

    @pl.when(step == 0)
    def _init():
        acc_ref[...] = jnp.zeros_like(acc_ref)
        inv_tau = jnp.float32(1.0 / _TAU)
        for b in range(_B):
            xcol = rt_ref[:, b : b + 1]
            rk = jnp.zeros((_N, 1), jnp.float32)
            for c in range(_N // _CH):
                xrow = r_ref[b : b + 1, c * _CH : (c + 1) * _CH]
                d = (xcol - xrow) * inv_tau
                rk = rk + jnp.sum(jax.nn.sigmoid(d), axis=1, keepdims=True)
            rank = rk + 0.5
            wt_ref[:, b : b + 1] = jax.nn.sigmoid(rank - (_N - _TOP_K))
        iota = jax.lax.broadcasted_iota(jnp.int32, (_N, _B), 0)
        neg = jnp.float32(-jnp.inf)

        def kstep(_, carry):
            xm, ind = carry
            mx = jnp.max(xm, axis=0, keepdims=True)
            hit = xm == mx
            first = jnp.min(jnp.where(hit, iota, _N), axis=0, keepdims=True)
            oh = (iota == first).astype(jnp.float32)
            return jnp.where(oh > 0, neg, xm), ind + oh

        xm0 = rt_ref[...]
        ind0 = jnp.zeros((_N, _B), jnp.float32)
        _, ind = jax.lax.fori_loop(0, _TOP_K, kstep, (xm0, ind0))
        wt_ref[:, _B:] = ind

    wfull = wt_ref[...]
    p = jnp.dot(m_ref[...], wfull, preferred_element_type=jnp.float32)
    wblk = wt_ref[pl.ds(step * _BLK, _BLK), :]
    acc_ref[...] += jnp.sum(wblk * p, axis=0, keepdims=True)

    @pl.when(step == _STEPS - 1)
    def _final():
        w = wt_ref[...]
        ws = jnp.sum(w, axis=0, keepdims=True)
        wss = jnp.sum(w * w, axis=0, keepdims=True)
        den = ws * ws - wss
        avg = acc_ref[...] / (den + _EPS)
        out_ref[...] = jnp.where(den == 0, 0.0, avg)


def kernel(R, dist_mat):
    RT = R.T
    out = pl.pallas_call(
        _body,
        grid=(_STEPS,),
        in_specs=[
            pl.BlockSpec((_B, _N), lambda i: (0, 0)),
            pl.BlockSpec((_N, _B), lambda i: (0, 0)),
            pl.BlockSpec((_BLK, _N), lambda i: (i, 0)),
        ],
        out_specs=pl.BlockSpec((1, 2 * _B), lambda i: (0, 0)),
        out_shape=jax.ShapeDtypeStruct((1, 2 * _B), jnp.float32),
        scratch_shapes=[
            pltpu.VMEM((_N, 2 * _B), jnp.float32),
            pltpu.VMEM((1, 2 * _B), jnp.float32),
        ],
    )(R, RT, dist_mat)
    return out[0, :_B], out[0, _B:]


# speedup vs baseline: 5.0493x; 5.0493x over previous
import jax
import jax.numpy as jnp
from jax.experimental import pallas as pl
from jax.experimental.pallas import tpu as pltpu

_TOP_K = 10
_TAU = 1e-4
_EPS = 1e-8
_N = 2048
_B = 8
_BLK = 256
_STEPS = _N // _BLK
_CAP = 64
_TGT = 48
_BS_ITERS = 30


def _shift_right(p, sh):
    z = jnp.zeros((p.shape[0], sh), p.dtype)
    return jnp.concatenate([z, p[:, : p.shape[1] - sh]], axis=1)


def _body(r_ref, m_ref, out_ref, w_ref, acc_ref):
    step = pl.program_id(0)

    @pl.when(step == 0)
    def _init():
        acc_ref[...] = jnp.zeros_like(acc_ref)
        inv_tau = jnp.float32(1.0 / _TAU)
        x = r_ref[...]

        lo = jnp.min(x, axis=1, keepdims=True) - 1.0
        hi = jnp.max(x, axis=1, keepdims=True)

        def bs(_, carry):
            lo, hi = carry
            mid = 0.5 * (lo + hi)
            cnt = jnp.sum((x > mid).astype(jnp.float32), axis=1, keepdims=True)
            pred = cnt >= _TGT
            return jnp.where(pred, mid, lo), jnp.where(pred, hi, mid)

        lo, hi = jax.lax.fori_loop(0, _BS_ITERS, bs, (lo, hi))
        mask = (x > lo).astype(jnp.float32)

        p = mask
        sh = 1
        while sh < _N:
            p = p + _shift_right(p, sh)
            sh *= 2
        cnt_all = p[:, _N - 1 : _N]

        iota_c = jax.lax.broadcasted_iota(
            jnp.int32, (_CAP, 1), 0).astype(jnp.float32)
        neg = jnp.float32(-jnp.inf)
        for b in range(_B):
            prow = p[b : b + 1, :]
            mrow = mask[b : b + 1, :]
            bt = jnp.where((iota_c == prow - 1.0) & (mrow > 0), 1.0, 0.0)
            cand = jnp.sum(bt * r_ref[b : b + 1, :], axis=1, keepdims=True)
            d = (cand - r_ref[b : b + 1, :]) * inv_tau
            rank = jnp.sum(jax.nn.sigmoid(d), axis=1, keepdims=True) + 0.5
            valid = iota_c < cnt_all[b, 0]
            wcol = jnp.where(valid, jax.nn.sigmoid(rank - (_N - _TOP_K)), 0.0)

            cvals = jnp.where(valid, cand, neg)

            def kstep(_, carry):
                vals, sel = carry
                mx = jnp.max(vals, axis=0, keepdims=True)
                hit = vals == mx
                first = jnp.min(
                    jnp.where(hit, iota_c, _CAP), axis=0, keepdims=True)
                oh = jnp.where(iota_c == first, 1.0, 0.0)
                return jnp.where(oh > 0, neg, vals), sel + oh

            _, sel = jax.lax.fori_loop(
                0, _TOP_K, kstep, (cvals, jnp.zeros((_CAP, 1), jnp.float32)))

            w_ref[b : b + 1, :] = jnp.sum(wcol * bt, axis=0, keepdims=True)
            w_ref[_B + b : _B + b + 1, :] = jnp.sum(
                sel * bt, axis=0, keepdims=True)

    wblk = w_ref[:, pl.ds(step * _BLK, _BLK)]
    acc_ref[...] += jnp.dot(
        wblk, m_ref[...], preferred_element_type=jnp.float32)

    @pl.when(step == _STEPS - 1)
    def _final():
        w = w_ref[...]
        num = jnp.sum(acc_ref[...] * w, axis=1, keepdims=True)
        ws = jnp.sum(w, axis=1, keepdims=True)
        wss = jnp.sum(w * w, axis=1, keepdims=True)
        den = ws * ws - wss
        avg = num / (den + _EPS)
        avg = jnp.where(den == 0, 0.0, avg)
        out_ref[...] = jnp.broadcast_to(avg, (2 * _B, 128))


def kernel(R, dist_mat):
    out = pl.pallas_call(
        _body,
        grid=(_STEPS,),
        in_specs=[
            pl.BlockSpec((_B, _N), lambda i: (0, 0)),
            pl.BlockSpec((_BLK, _N), lambda i: (i, 0)),
        ],
        out_specs=pl.BlockSpec((2 * _B, 128), lambda i: (0, 0)),
        out_shape=jax.ShapeDtypeStruct((2 * _B, 128), jnp.float32),
        scratch_shapes=[
            pltpu.VMEM((2 * _B, _N), jnp.float32),
            pltpu.VMEM((2 * _B, _N), jnp.float32),
        ],
    )(R, dist_mat)
    return out[:_B, 0], out[_B:, 0]


# speedup vs baseline: 8.6481x; 1.7127x over previous
import jax
import jax.numpy as jnp
from jax.experimental import pallas as pl
from jax.experimental.pallas import tpu as pltpu

_TOP_K = 10
_TAU = 1e-4
_EPS = 1e-8
_N = 2048
_B = 8
_CAP = 64
_TGT = 48
_BS_ITERS = 16


def _shift_right(p, sh):
    z = jnp.zeros((p.shape[0], sh), p.dtype)
    return jnp.concatenate([z, p[:, : p.shape[1] - sh]], axis=1)


def _body(r_ref, m_hbm, out_ref, m_vmem, sem):
    cp = pltpu.make_async_copy(m_hbm, m_vmem, sem)
    cp.start()

    inv_tau = jnp.float32(1.0 / _TAU)
    x = r_ref[...]

    lo = jnp.min(x, axis=1, keepdims=True) - 1.0
    hi = jnp.max(x, axis=1, keepdims=True)

    def bs(_, carry):
        lo, hi = carry
        mid = 0.5 * (lo + hi)
        cnt = jnp.sum((x > mid).astype(jnp.float32), axis=1, keepdims=True)
        pred = cnt >= _TGT
        return jnp.where(pred, mid, lo), jnp.where(pred, hi, mid)

    lo, hi = jax.lax.fori_loop(0, _BS_ITERS, bs, (lo, hi))
    mask = (x > lo).astype(jnp.float32)

    p = mask
    sh = 1
    while sh < _N:
        p = p + _shift_right(p, sh)
        sh *= 2
    cnt_all = p[:, _N - 1 : _N]

    big = (_B * _CAP, _N)
    x_rep = jnp.concatenate(
        [jnp.broadcast_to(x[b : b + 1, :], (_CAP, _N)) for b in range(_B)], 0)
    p_rep = jnp.concatenate(
        [jnp.broadcast_to(p[b : b + 1, :], (_CAP, _N)) for b in range(_B)], 0)
    m_rep = jnp.concatenate(
        [jnp.broadcast_to(mask[b : b + 1, :], (_CAP, _N)) for b in range(_B)],
        0)
    cnt_rep = jnp.concatenate(
        [jnp.broadcast_to(cnt_all[b : b + 1, :], (_CAP, 1)) for b in range(_B)],
        0)
    iota_r = jax.lax.broadcasted_iota(jnp.int32, (big[0], 1), 0)
    slot = (iota_r - (iota_r >> 6 << 6)).astype(jnp.float32)
    bt = jnp.where((p_rep - 1.0 == slot) & (m_rep > 0), 1.0, 0.0)
    cand = jnp.sum(bt * x_rep, axis=1, keepdims=True)
    d = (cand - x_rep) * inv_tau
    rank = jnp.sum(jax.nn.sigmoid(d), axis=1, keepdims=True) + 0.5
    valid = slot < cnt_rep
    wcol = jnp.where(valid, jax.nn.sigmoid(rank - (_N - _TOP_K)), 0.0)

    r_iota = jax.lax.broadcasted_iota(jnp.int32, (_B, _B * _CAP), 0)
    c_iota = jax.lax.broadcasted_iota(jnp.int32, (_B, _B * _CAP), 1)
    s_mat = jnp.where(c_iota >> 6 == r_iota, 1.0, 0.0)
    w_apx = jnp.dot(s_mat, wcol * bt, preferred_element_type=jnp.float32)

    iota_n = jax.lax.broadcasted_iota(jnp.int32, (_B, _N), 1)
    neg = jnp.float32(-jnp.inf)

    def kstep(_, carry):
        vals, ind = carry
        mx = jnp.max(vals, axis=1, keepdims=True)
        hit = vals == mx
        first = jnp.min(jnp.where(hit, iota_n, _N), axis=1, keepdims=True)
        oh = jnp.where(iota_n == first, 1.0, 0.0)
        return jnp.where(oh > 0, neg, vals), ind + oh

    _, ind = jax.lax.fori_loop(
        0, _TOP_K, kstep, (x, jnp.zeros((_B, _N), jnp.float32)))

    w = jnp.concatenate([w_apx, ind], axis=0)

    cp.wait()
    acc = jnp.dot(w, m_vmem[...], preferred_element_type=jnp.float32)
    num = jnp.sum(acc * w, axis=1, keepdims=True)
    ws = jnp.sum(w, axis=1, keepdims=True)
    wss = jnp.sum(w * w, axis=1, keepdims=True)
    den = ws * ws - wss
    avg = num / (den + _EPS)
    avg = jnp.where(den == 0, 0.0, avg)
    out_ref[...] = jnp.broadcast_to(avg, (2 * _B, 128))


def kernel(R, dist_mat):
    out = pl.pallas_call(
        _body,
        in_specs=[
            pl.BlockSpec((_B, _N), lambda: (0, 0)),
            pl.BlockSpec(memory_space=pl.ANY),
        ],
        out_specs=pl.BlockSpec((2 * _B, 128), lambda: (0, 0)),
        out_shape=jax.ShapeDtypeStruct((2 * _B, 128), jnp.float32),
        scratch_shapes=[
            pltpu.VMEM((_N, _N), jnp.float32),
            pltpu.SemaphoreType.DMA,
        ],
    )(R, dist_mat)
    return out[:_B, 0], out[_B:, 0]
